# speedup vs baseline: 5.3585x; 5.3585x over previous
import functools

import jax
import jax.numpy as jnp
from jax import lax
from jax.experimental import pallas as pl
from jax.experimental.pallas import tpu as pltpu
from jax.experimental.pallas import tpu_sc as plsc

N_NODES_C = 10000
D = 128

NC = 2
NS = 16
K = 128
NCHUNK = 79
PE = NC * NS * NCHUNK * K
PADN = 10240
RPT = PADN // NS


def _scale_rows_body(x_ref, s_ref, o_ref):
    o_ref[...] = x_ref[...] * s_ref[...]


def _scale_rows(x, s):
    n = x.shape[0]
    blk = 2000
    grid = n // blk
    return pl.pallas_call(
        _scale_rows_body,
        grid=(grid,),
        in_specs=[
            pl.BlockSpec((blk, D), lambda i: (i, 0)),
            pl.BlockSpec((blk, 1), lambda i: (i, 0)),
        ],
        out_specs=pl.BlockSpec((blk, D), lambda i: (i, 0)),
        out_shape=jax.ShapeDtypeStruct((n, D), jnp.float32),
    )(x, s)


def _combine_body(a_ref, b_ref, s_ref, o_ref):
    o_ref[...] = (a_ref[...] + b_ref[...]) * s_ref[...]


def _combine(a, b, s):
    n = a.shape[0]
    blk = 2000
    grid = n // blk
    return pl.pallas_call(
        _combine_body,
        grid=(grid,),
        in_specs=[
            pl.BlockSpec((blk, D), lambda i: (i, 0)),
            pl.BlockSpec((blk, D), lambda i: (i, 0)),
            pl.BlockSpec((blk, 1), lambda i: (i, 0)),
        ],
        out_specs=pl.BlockSpec((blk, D), lambda i: (i, 0)),
        out_shape=jax.ShapeDtypeStruct((n, D), jnp.float32),
    )(a, b, s)


def _sc_body(h_hbm, src_hbm, dst_hbm, z_hbm, out_hbm,
             src_v, dst_v, rows_v, acc, sem):
    c = lax.axis_index("c")
    s = lax.axis_index("s")
    pltpu.sync_copy(src_hbm.at[c, s], src_v)
    pltpu.sync_copy(dst_hbm.at[c, s], dst_v)
    pltpu.sync_copy(z_hbm, acc.at[pl.ds(s * RPT, RPT)])
    plsc.subcore_barrier()

    def chunk(j, carry):
        pltpu.async_copy(h_hbm.at[src_v.at[j]], rows_v, sem).wait()
        pltpu.sync_copy(rows_v, acc.at[dst_v.at[j]], add=True)
        return carry

    lax.fori_loop(0, NCHUNK, chunk, 0)
    plsc.subcore_barrier()
    pltpu.sync_copy(acc.at[pl.ds(s * RPT, RPT)],
                    out_hbm.at[c, pl.ds(s * RPT, RPT)])


@functools.partial(
    pl.kernel,
    mesh=plsc.VectorSubcoreMesh(core_axis_name="c", subcore_axis_name="s"),
    out_type=jax.ShapeDtypeStruct((NC, PADN, D), jnp.float32),
    scratch_types=[
        pltpu.VMEM((NCHUNK, K), jnp.int32),
        pltpu.VMEM((NCHUNK, K), jnp.int32),
        pltpu.VMEM((K, D), jnp.float32),
        pltpu.VMEM_SHARED((PADN, D), jnp.float32),
        pltpu.SemaphoreType.DMA,
    ],
)
def _sc_scatter(h_hbm, src_hbm, dst_hbm, z_hbm, out_hbm,
                src_v, dst_v, rows_v, acc, sem):
    _sc_body(h_hbm, src_hbm, dst_hbm, z_hbm, out_hbm,
             src_v, dst_v, rows_v, acc, sem)


def kernel(feat, edge_index, cj, ci, weight):
    n = feat.shape[0]
    src = edge_index[0].astype(jnp.int32)
    dst = edge_index[1].astype(jnp.int32)

    h = _scale_rows(feat, cj)

    pad = PE - src.shape[0]
    src_p = jnp.concatenate(
        [src, jnp.zeros((pad,), jnp.int32)]).reshape(NC, NS, NCHUNK, K)
    dst_p = jnp.concatenate(
        [dst, jnp.full((pad,), PADN - 1, jnp.int32)]).reshape(NC, NS, NCHUNK, K)
    zeros = jnp.zeros((RPT, D), jnp.float32)

    partial = _sc_scatter(h, src_p, dst_p, zeros)
    return _combine(partial[0, :n], partial[1, :n], ci)
